# Initial kernel scaffold; baseline (speedup 1.0000x reference)
import jax
import jax.numpy as jnp
from jax import lax
from jax.experimental import pallas as pl
from jax.experimental.pallas import tpu as pltpu
from jax.experimental.pallas import tpu_sc as plsc

MAXLEN = 200
EMBED = 32
BATCH = 4096

NC, NS = 2, 16
NW = NC * NS
ROWS = BATCH * MAXLEN
CHUNK = 1600
GROWS = 100
GPC = CHUNK // GROWS
NCHUNKS = ROWS // CHUNK
CPW = NCHUNKS // NW


def _emb_body(x3, table, pos, out, idx_v, rows_v, pos_v, sem):
    wid = lax.axis_index("s") * NC + lax.axis_index("c")
    pltpu.sync_copy(pos, pos_v)

    def chunk_body(g, carry):
        c = wid * CPW + g
        pltpu.sync_copy(x3.at[c], idx_v)
        copies = [
            pltpu.async_copy(
                table.at[idx_v.at[j]],
                rows_v.at[pl.ds(j * GROWS, GROWS)],
                sem,
            )
            for j in range(GPC)
        ]
        for cp in copies:
            cp.wait()

        def r_body(r, carry2):
            def q_body(q, carry3):
                row = r * MAXLEN + q
                for h in (0, 16):
                    rows_v[row, pl.ds(h, 16)] = (
                        rows_v[row, pl.ds(h, 16)] + pos_v[q, pl.ds(h, 16)]
                    )
                return carry3

            return lax.fori_loop(0, MAXLEN, q_body, carry2)

        lax.fori_loop(0, CHUNK // MAXLEN, r_body, 0)

        pltpu.sync_copy(rows_v, out.at[pl.ds(c * CHUNK, CHUNK)])
        return carry

    lax.fori_loop(0, CPW, chunk_body, 0)


@jax.jit
def kernel(x, token_table, pos_table):
    x3 = x.astype(jnp.int32).reshape(NCHUNKS, GPC, GROWS)
    mesh = plsc.VectorSubcoreMesh(core_axis_name="c", subcore_axis_name="s")
    out = pl.kernel(
        _emb_body,
        out_type=jax.ShapeDtypeStruct((ROWS, EMBED), jnp.float32),
        mesh=mesh,
        scratch_types=[
            pltpu.VMEM((GPC, GROWS), jnp.int32),
            pltpu.VMEM((CHUNK, EMBED), jnp.float32),
            pltpu.VMEM((MAXLEN, EMBED), jnp.float32),
            pltpu.SemaphoreType.DMA,
        ],
    )(x3, token_table, pos_table)
    return out.reshape(BATCH, MAXLEN, EMBED)


# speedup vs baseline: 1.3188x; 1.3188x over previous
import jax
import jax.numpy as jnp
from jax import lax
from jax.experimental import pallas as pl
from jax.experimental.pallas import tpu as pltpu
from jax.experimental.pallas import tpu_sc as plsc

MAXLEN = 200
EMBED = 32
BATCH = 4096

NC, NS = 2, 16
NW = NC * NS
ROWS = BATCH * MAXLEN
CHUNK = 1600
GROWS = 100
GPC = CHUNK // GROWS
NCHUNKS = ROWS // CHUNK
CPW = NCHUNKS // NW


def _emb_body(x3, table, pos, out, idx_v, rows_v, pos_v, sem):
    wid = lax.axis_index("s") * NC + lax.axis_index("c")
    pltpu.sync_copy(pos, pos_v)

    def chunk_body(g, carry):
        c = wid * CPW + g
        pltpu.sync_copy(x3.at[c], idx_v)
        copies = [
            pltpu.async_copy(
                table.at[idx_v.at[j]],
                rows_v.at[pl.ds(j * GROWS, GROWS)],
                sem,
            )
            for j in range(GPC)
        ]
        for cp in copies:
            cp.wait()

        def r_body(r, carry2):
            def q_body(q, carry3):
                row = r * MAXLEN + q
                for h in (0, 16):
                    rows_v[row, pl.ds(h, 16)] = (
                        rows_v[row, pl.ds(h, 16)] + pos_v[q, pl.ds(h, 16)]
                    )
                return carry3

            return lax.fori_loop(0, MAXLEN, q_body, carry2)

        lax.fori_loop(0, CHUNK // MAXLEN, r_body, 0)

        pltpu.sync_copy(rows_v, out.at[pl.ds(c * CHUNK, CHUNK)])
        return carry

    lax.fori_loop(0, CPW, chunk_body, 0)


@jax.jit
def kernel(x, token_table, pos_table):
    x3 = x.astype(jnp.int32).reshape(NCHUNKS, GPC, GROWS)
    mesh = plsc.VectorSubcoreMesh(core_axis_name="c", subcore_axis_name="s")
    out = pl.kernel(
        _emb_body,
        out_type=jax.ShapeDtypeStruct((ROWS, EMBED), jnp.float32),
        mesh=mesh,
        compiler_params=pltpu.CompilerParams(use_tc_tiling_on_sc=False),
        scratch_types=[
            pltpu.VMEM((GPC, GROWS), jnp.int32),
            pltpu.VMEM((CHUNK, EMBED), jnp.float32),
            pltpu.VMEM((MAXLEN, EMBED), jnp.float32),
            pltpu.SemaphoreType.DMA,
        ],
    )(x3, token_table, pos_table)
    return out.reshape(BATCH, MAXLEN, EMBED)


# speedup vs baseline: 1.4256x; 1.0810x over previous
import jax
import jax.numpy as jnp
from jax import lax
from jax.experimental import pallas as pl
from jax.experimental.pallas import tpu as pltpu
from jax.experimental.pallas import tpu_sc as plsc

MAXLEN = 200
EMBED = 32
BATCH = 4096

NC, NS = 2, 16
NW = NC * NS
ROWS = BATCH * MAXLEN
CHUNK = 1600
GROWS = 100
GPC = CHUNK // GROWS
NCHUNKS = ROWS // CHUNK
CPW = NCHUNKS // NW


def _emb_body(x3, table, pos, out, idx_v, rows_v, pos_v, sem):
    wid = lax.axis_index("s") * NC + lax.axis_index("c")
    pltpu.sync_copy(pos, pos_v)

    def chunk_body(g, carry):
        c = wid * CPW + g
        pltpu.sync_copy(x3.at[c], idx_v)
        copies = [
            pltpu.async_copy(
                table.at[idx_v.at[j]],
                rows_v.at[pl.ds(j * GROWS, GROWS)],
                sem,
            )
            for j in range(GPC)
        ]
        for cp in copies:
            cp.wait()

        def q_body(q, carry2):
            for h in (0, 16):
                p = pos_v[q, pl.ds(h, 16)]
                for r in range(CHUNK // MAXLEN):
                    row = r * MAXLEN + q
                    rows_v[row, pl.ds(h, 16)] = rows_v[row, pl.ds(h, 16)] + p
            return carry2

        lax.fori_loop(0, MAXLEN, q_body, 0)

        pltpu.sync_copy(rows_v, out.at[pl.ds(c * CHUNK, CHUNK)])
        return carry

    lax.fori_loop(0, CPW, chunk_body, 0)


@jax.jit
def kernel(x, token_table, pos_table):
    x3 = x.astype(jnp.int32).reshape(NCHUNKS, GPC, GROWS)
    mesh = plsc.VectorSubcoreMesh(core_axis_name="c", subcore_axis_name="s")
    out = pl.kernel(
        _emb_body,
        out_type=jax.ShapeDtypeStruct((ROWS, EMBED), jnp.float32),
        mesh=mesh,
        compiler_params=pltpu.CompilerParams(use_tc_tiling_on_sc=False),
        scratch_types=[
            pltpu.VMEM((GPC, GROWS), jnp.int32),
            pltpu.VMEM((CHUNK, EMBED), jnp.float32),
            pltpu.VMEM((MAXLEN, EMBED), jnp.float32),
            pltpu.SemaphoreType.DMA,
        ],
    )(x3, token_table, pos_table)
    return out.reshape(BATCH, MAXLEN, EMBED)
